# baseline (speedup 1.0000x reference)
import functools

import jax
import jax.numpy as jnp
from jax import lax
from jax.experimental import pallas as pl
from jax.experimental.pallas import tpu as pltpu
from jax.experimental.pallas import tpu_sc as plsc

B = 16384
D = 64
NC = 2
NS = 16
NW = NC * NS
BW = B // NW

_mesh = plsc.VectorSubcoreMesh(core_axis_name="c", subcore_axis_name="s")


@functools.partial(
    pl.kernel,
    out_type=(
        jax.ShapeDtypeStruct((B, D), jnp.float32),
        jax.ShapeDtypeStruct((B, D), jnp.float32),
        jax.ShapeDtypeStruct((B, D), jnp.float32),
    ),
    mesh=_mesh,
    compiler_params=pltpu.CompilerParams(use_tc_tiling_on_sc=False),
    scratch_types=[
        pltpu.VMEM((BW,), jnp.int32),
        pltpu.VMEM((BW,), jnp.int32),
        pltpu.VMEM((BW,), jnp.int32),
        pltpu.VMEM((BW, D), jnp.float32),
        pltpu.VMEM((BW, D), jnp.float32),
        pltpu.VMEM((BW, D), jnp.float32),
        pltpu.SemaphoreType.DMA,
        pltpu.SemaphoreType.DMA,
    ],
)
def _sc_gather(u_idx_hbm, p_idx_hbm, n_idx_hbm, user_hbm, item_hbm,
               u_out, p_out, n_out,
               iu_v, ip_v, in_v, ru_v, rp_v, rn_v, gsem, wsem):
    wid = lax.axis_index("s") * NC + lax.axis_index("c")
    base = wid * BW
    sl = pl.ds(base, BW)
    pltpu.sync_copy(u_idx_hbm.at[sl], iu_v)
    pltpu.sync_copy(p_idx_hbm.at[sl], ip_v)
    pltpu.sync_copy(n_idx_hbm.at[sl], in_v)
    g0 = pltpu.async_copy(user_hbm.at[iu_v], ru_v, gsem)
    g1 = pltpu.async_copy(item_hbm.at[ip_v], rp_v, gsem)
    g2 = pltpu.async_copy(item_hbm.at[in_v], rn_v, gsem)
    g0.wait()
    w0 = pltpu.async_copy(ru_v, u_out.at[sl], wsem)
    g1.wait()
    w1 = pltpu.async_copy(rp_v, p_out.at[sl], wsem)
    g2.wait()
    w2 = pltpu.async_copy(rn_v, n_out.at[sl], wsem)
    w0.wait()
    w1.wait()
    w2.wait()


_TC_BLK = 2048


def _tc_dist_body(u_ref, i_ref, j_ref, pos_ref, neg_ref):
    u = u_ref[...]
    i = i_ref[...]
    j = j_ref[...]
    ssq_u = jnp.sum(u * u, axis=1, keepdims=True)
    ssq_i = jnp.sum(i * i, axis=1, keepdims=True)
    ssq_j = jnp.sum(j * j, axis=1, keepdims=True)
    dot_i = jnp.sum(u * i, axis=1, keepdims=True)
    dot_j = jnp.sum(u * j, axis=1, keepdims=True)
    mu = jnp.maximum(ssq_u, 1.0)
    mi = jnp.maximum(ssq_i, 1.0)
    mj = jnp.maximum(ssq_j, 1.0)
    pos_ref[...] = ssq_u / mu + ssq_i / mi - 2.0 * dot_i * lax.rsqrt(mu * mi)
    neg_ref[...] = ssq_u / mu + ssq_j / mj - 2.0 * dot_j * lax.rsqrt(mu * mj)


def _tc_dist(u_rows, p_rows, n_rows):
    row_spec = pl.BlockSpec((_TC_BLK, D), lambda b: (b, 0))
    out_spec = pl.BlockSpec((_TC_BLK, 1), lambda b: (b, 0))
    return pl.pallas_call(
        _tc_dist_body,
        grid=(B // _TC_BLK,),
        in_specs=[row_spec, row_spec, row_spec],
        out_specs=[out_spec, out_spec],
        out_shape=[
            jax.ShapeDtypeStruct((B, 1), jnp.float32),
            jax.ShapeDtypeStruct((B, 1), jnp.float32),
        ],
    )(u_rows, p_rows, n_rows)


def kernel(batch_user, batch_pos_item, batch_neg_item, user_emb, item_emb):
    u_rows, p_rows, n_rows = _sc_gather(
        batch_user, batch_pos_item, batch_neg_item, user_emb, item_emb)
    pos, neg = _tc_dist(u_rows, p_rows, n_rows)
    return (pos, neg)
